# Initial kernel scaffold; baseline (speedup 1.0000x reference)
import jax
import jax.numpy as jnp
from jax.experimental import pallas as pl


def kernel(x, gamma):
    raise NotImplementedError("write your pallas kernel here")


# speedup vs baseline: 1.4500x; 1.4500x over previous
import jax
import jax.numpy as jnp
from jax import lax
from jax.experimental import pallas as pl

BS = 7
H = 224
W = 224
HM = H - (BS - 1)
WM = W - (BS - 1)
PLANES = 384
COUNT_M = float(PLANES * H * W)
P1 = 8
P2 = 8


def _threefry_bits(idx):
    ks0 = jnp.int32(0)
    ks1 = jnp.int32(42)
    ks2 = jnp.int32(42 ^ 0x1BD11BDA)
    x0 = jnp.zeros_like(idx)
    x1 = idx + ks1

    def rotl(v, d):
        return lax.shift_left(v, jnp.int32(d)) | lax.shift_right_logical(
            v, jnp.int32(32 - d)
        )

    rot_a = (13, 15, 26, 6)
    rot_b = (17, 29, 16, 24)
    inject = ((ks1, ks2, 1), (ks2, ks0, 2), (ks0, ks1, 3), (ks1, ks2, 4),
              (ks2, ks0, 5))
    for g in range(5):
        for d in rot_a if g % 2 == 0 else rot_b:
            x0 = x0 + x1
            x1 = rotl(x1, d)
            x1 = x1 ^ x0
        a, b, inc = inject[g]
        x0 = x0 + a
        x1 = x1 + b + jnp.int32(inc)
    return x0 ^ x1


def _phase1_kernel(gamma_ref, x_ref, y_ref, cnt_ref):
    k = pl.program_id(0)
    gamma = gamma_ref[...]
    p = lax.broadcasted_iota(jnp.int32, (P1, H, W), 0)
    r = lax.broadcasted_iota(jnp.int32, (P1, H, W), 1)
    c = lax.broadcasted_iota(jnp.int32, (P1, H, W), 2)
    idx = (k * P1 + p) * (HM * WM) + r * WM + c
    bits = _threefry_bits(idx)
    f = lax.bitcast_convert_type(
        lax.shift_right_logical(bits, jnp.int32(9)) | jnp.int32(0x3F800000),
        jnp.float32,
    )
    u = f - jnp.float32(1.0)
    valid = (r < HM) & (c < WM)
    seed = jnp.where(valid & (u < gamma), jnp.float32(1.0), jnp.float32(0.0))
    t = seed
    for s in range(1, BS):
        t = jnp.maximum(t, jnp.roll(seed, s, axis=1))
    d = t
    for s in range(1, BS):
        d = jnp.maximum(d, jnp.roll(t, s, axis=2))
    block_mask = jnp.float32(1.0) - d
    y_ref[...] = block_mask * x_ref[...]
    ones = jnp.sum(block_mask, keepdims=True).reshape(1, 1)

    @pl.when(k == 0)
    def _():
        cnt_ref[...] = jnp.zeros((1, 1), jnp.float32)

    cnt_ref[...] += ones


def _phase2_kernel(cnt_ref, y_ref, out_ref):
    scale = jnp.float32(COUNT_M) / cnt_ref[...]
    out_ref[...] = y_ref[...] * scale


def kernel(x, gamma):
    b, ch, h, w = x.shape
    xf = x.reshape(b * ch, h, w)
    gamma_arr = jnp.asarray(gamma, jnp.float32).reshape(1, 1)
    y, cnt = pl.pallas_call(
        _phase1_kernel,
        grid=(PLANES // P1,),
        in_specs=[
            pl.BlockSpec((1, 1), lambda k: (0, 0)),
            pl.BlockSpec((P1, H, W), lambda k: (k, 0, 0)),
        ],
        out_specs=[
            pl.BlockSpec((P1, H, W), lambda k: (k, 0, 0)),
            pl.BlockSpec((1, 1), lambda k: (0, 0)),
        ],
        out_shape=[
            jax.ShapeDtypeStruct((PLANES, H, W), jnp.float32),
            jax.ShapeDtypeStruct((1, 1), jnp.float32),
        ],
    )(gamma_arr, xf)
    out = pl.pallas_call(
        _phase2_kernel,
        grid=(PLANES // P2,),
        in_specs=[
            pl.BlockSpec((1, 1), lambda k: (0, 0)),
            pl.BlockSpec((P2, H, W), lambda k: (k, 0, 0)),
        ],
        out_specs=pl.BlockSpec((P2, H, W), lambda k: (k, 0, 0)),
        out_shape=jax.ShapeDtypeStruct((PLANES, H, W), jnp.float32),
    )(cnt, y)
    return out.reshape(b, ch, h, w)


# speedup vs baseline: 1.9929x; 1.3744x over previous
import jax
import jax.numpy as jnp
from jax import lax
from jax.experimental import pallas as pl

BS = 7
H = 224
W = 224
HM = H - (BS - 1)
WM = W - (BS - 1)
PLANES = 384
COUNT_M = float(PLANES * H * W)
P1 = 8
P2 = 8


def _threefry_bits(x1):
    ks0 = jnp.int32(0)
    ks1 = jnp.int32(42)
    ks2 = jnp.int32(42 ^ 0x1BD11BDA)
    x0 = jnp.zeros_like(x1)

    def rotl(v, d):
        return lax.shift_left(v, jnp.int32(d)) | lax.shift_right_logical(
            v, jnp.int32(32 - d)
        )

    rot_a = (13, 15, 26, 6)
    rot_b = (17, 29, 16, 24)
    inject = ((ks1, ks2, 1), (ks2, ks0, 2), (ks0, ks1, 3), (ks1, ks2, 4),
              (ks2, ks0, 5))
    for g in range(5):
        for d in rot_a if g % 2 == 0 else rot_b:
            x0 = x0 + x1
            x1 = rotl(x1, d)
            x1 = x1 ^ x0
        a, b, inc = inject[g]
        x0 = x0 + a
        x1 = x1 + b + jnp.int32(inc)
    return x0 ^ x1


def _phase1_kernel(gamma_ref, idx_ref, vmask_ref, lb_ref, rb_ref, x_ref,
                   y_ref, cnt_ref):
    k = pl.program_id(0)
    gamma = gamma_ref[...]
    base = k * jnp.int32(P1 * HM * WM)
    bits = _threefry_bits(idx_ref[...] + base)
    u = lax.bitcast_convert_type(
        lax.shift_right_logical(bits, jnp.int32(9)) | jnp.int32(0x3F800000),
        jnp.float32,
    ) - jnp.float32(1.0)
    seed = jnp.where(u < gamma, vmask_ref[...], jnp.float32(0.0))
    sb = seed.astype(jnp.bfloat16)
    lb = lb_ref[...]
    rb = rb_ref[...]
    ones = None
    for p in range(P1):
        t = jnp.dot(lb, sb[p], preferred_element_type=jnp.float32)
        d = jnp.dot(t.astype(jnp.bfloat16), rb,
                    preferred_element_type=jnp.float32)
        bm = jnp.where(d < jnp.float32(0.5), jnp.float32(1.0), jnp.float32(0.0))
        y_ref[p, :, :] = bm * x_ref[p, :, :]
        s = jnp.sum(bm, keepdims=True).reshape(1, 1)
        ones = s if ones is None else ones + s

    @pl.when(k == 0)
    def _():
        cnt_ref[...] = jnp.zeros((1, 1), jnp.float32)

    cnt_ref[...] += ones


def _phase2_kernel(cnt_ref, y_ref, out_ref):
    scale = jnp.float32(COUNT_M) / cnt_ref[...]
    out_ref[...] = y_ref[...] * scale


def kernel(x, gamma):
    b, ch, h, w = x.shape
    xf = x.reshape(b * ch, h, w)
    gamma_arr = jnp.asarray(gamma, jnp.float32).reshape(1, 1)

    pp = jnp.arange(P1, dtype=jnp.int32)[:, None, None]
    rr = jnp.arange(H, dtype=jnp.int32)[None, :, None]
    cc = jnp.arange(W, dtype=jnp.int32)[None, None, :]
    idx0 = pp * (HM * WM) + rr * WM + cc + 42
    vmask = jnp.where((rr < HM) & (cc < WM), jnp.float32(1.0),
                      jnp.float32(0.0)) + jnp.zeros((P1, 1, 1), jnp.float32)
    ii = jnp.arange(H, dtype=jnp.int32)
    band = ((ii[None, :] <= ii[:, None]) &
            (ii[:, None] - ii[None, :] <= BS - 1))
    lband = band.astype(jnp.bfloat16)
    rband = lband.T

    y, cnt = pl.pallas_call(
        _phase1_kernel,
        grid=(PLANES // P1,),
        in_specs=[
            pl.BlockSpec((1, 1), lambda k: (0, 0)),
            pl.BlockSpec((P1, H, W), lambda k: (0, 0, 0)),
            pl.BlockSpec((P1, H, W), lambda k: (0, 0, 0)),
            pl.BlockSpec((H, W), lambda k: (0, 0)),
            pl.BlockSpec((H, W), lambda k: (0, 0)),
            pl.BlockSpec((P1, H, W), lambda k: (k, 0, 0)),
        ],
        out_specs=[
            pl.BlockSpec((P1, H, W), lambda k: (k, 0, 0)),
            pl.BlockSpec((1, 1), lambda k: (0, 0)),
        ],
        out_shape=[
            jax.ShapeDtypeStruct((PLANES, H, W), jnp.float32),
            jax.ShapeDtypeStruct((1, 1), jnp.float32),
        ],
    )(gamma_arr, idx0, vmask, lband, rband, xf)
    out = pl.pallas_call(
        _phase2_kernel,
        grid=(PLANES // P2,),
        in_specs=[
            pl.BlockSpec((1, 1), lambda k: (0, 0)),
            pl.BlockSpec((P2, H, W), lambda k: (k, 0, 0)),
        ],
        out_specs=pl.BlockSpec((P2, H, W), lambda k: (k, 0, 0)),
        out_shape=jax.ShapeDtypeStruct((PLANES, H, W), jnp.float32),
    )(cnt, y)
    return out.reshape(b, ch, h, w)


# speedup vs baseline: 2.0425x; 1.0249x over previous
import jax
import jax.numpy as jnp
from jax import lax
from jax.experimental import pallas as pl

BS = 7
H = 224
W = 224
HM = H - (BS - 1)
WM = W - (BS - 1)
PLANES = 384
COUNT_M = float(PLANES * H * W)
P1 = 8
P2 = 32


def _threefry_bits(x1):
    ks0 = jnp.int32(0)
    ks1 = jnp.int32(42)
    ks2 = jnp.int32(42 ^ 0x1BD11BDA)
    x0 = jnp.zeros_like(x1)

    def rotl(v, d):
        return lax.shift_left(v, jnp.int32(d)) | lax.shift_right_logical(
            v, jnp.int32(32 - d)
        )

    rot_a = (13, 15, 26, 6)
    rot_b = (17, 29, 16, 24)
    inject = ((ks1, ks2, 1), (ks2, ks0, 2), (ks0, ks1, 3), (ks1, ks2, 4),
              (ks2, ks0, 5))
    for g in range(5):
        for d in rot_a if g % 2 == 0 else rot_b:
            x0 = x0 + x1
            x1 = rotl(x1, d)
            x1 = x1 ^ x0
        a, b, inc = inject[g]
        x0 = x0 + a
        x1 = x1 + b + jnp.int32(inc)
    return x0 ^ x1


def _phase1_kernel(gamma_ref, idx_ref, vmask_ref, lb_ref, rb_ref, x_ref,
                   y_ref, cnt_ref):
    k = pl.program_id(0)
    gamma = gamma_ref[...]
    base = k * jnp.int32(P1 * HM * WM)
    bits = _threefry_bits(idx_ref[...] + base)
    u = lax.bitcast_convert_type(
        lax.shift_right_logical(bits, jnp.int32(9)) | jnp.int32(0x3F800000),
        jnp.float32,
    ) - jnp.float32(1.0)
    seed = jnp.where(u < gamma, vmask_ref[...], jnp.float32(0.0))
    sb = seed.astype(jnp.bfloat16)
    lb = lb_ref[...]
    rb = rb_ref[...]
    ones = None
    for p in range(P1):
        t = jnp.dot(lb, sb[p], preferred_element_type=jnp.float32)
        d = jnp.dot(t.astype(jnp.bfloat16), rb,
                    preferred_element_type=jnp.float32)
        bm = jnp.where(d < jnp.float32(0.5), jnp.float32(1.0), jnp.float32(0.0))
        y_ref[p, :, :] = bm * x_ref[p, :, :]
        s = jnp.sum(bm, keepdims=True).reshape(1, 1)
        ones = s if ones is None else ones + s

    @pl.when(k == 0)
    def _():
        cnt_ref[...] = jnp.zeros((1, 1), jnp.float32)

    cnt_ref[...] += ones


def _phase2_kernel(cnt_ref, y_ref, out_ref):
    scale = jnp.float32(COUNT_M) / cnt_ref[...]
    out_ref[...] = y_ref[...] * scale


def kernel(x, gamma):
    b, ch, h, w = x.shape
    xf = x.reshape(b * ch, h, w)
    gamma_arr = jnp.asarray(gamma, jnp.float32).reshape(1, 1)

    pp = jnp.arange(P1, dtype=jnp.int32)[:, None, None]
    rr = jnp.arange(H, dtype=jnp.int32)[None, :, None]
    cc = jnp.arange(W, dtype=jnp.int32)[None, None, :]
    idx0 = pp * (HM * WM) + rr * WM + cc + 42
    vmask = jnp.where((rr < HM) & (cc < WM), jnp.float32(1.0),
                      jnp.float32(0.0)) + jnp.zeros((P1, 1, 1), jnp.float32)
    ii = jnp.arange(H, dtype=jnp.int32)
    band = ((ii[None, :] <= ii[:, None]) &
            (ii[:, None] - ii[None, :] <= BS - 1))
    lband = band.astype(jnp.bfloat16)
    rband = lband.T

    y, cnt = pl.pallas_call(
        _phase1_kernel,
        grid=(PLANES // P1,),
        in_specs=[
            pl.BlockSpec((1, 1), lambda k: (0, 0)),
            pl.BlockSpec((P1, H, W), lambda k: (0, 0, 0)),
            pl.BlockSpec((P1, H, W), lambda k: (0, 0, 0)),
            pl.BlockSpec((H, W), lambda k: (0, 0)),
            pl.BlockSpec((H, W), lambda k: (0, 0)),
            pl.BlockSpec((P1, H, W), lambda k: (k, 0, 0)),
        ],
        out_specs=[
            pl.BlockSpec((P1, H, W), lambda k: (k, 0, 0)),
            pl.BlockSpec((1, 1), lambda k: (0, 0)),
        ],
        out_shape=[
            jax.ShapeDtypeStruct((PLANES, H, W), jnp.float32),
            jax.ShapeDtypeStruct((1, 1), jnp.float32),
        ],
    )(gamma_arr, idx0, vmask, lband, rband, xf)
    out = pl.pallas_call(
        _phase2_kernel,
        grid=(PLANES // P2,),
        in_specs=[
            pl.BlockSpec((1, 1), lambda k: (0, 0)),
            pl.BlockSpec((P2, H, W), lambda k: (k, 0, 0)),
        ],
        out_specs=pl.BlockSpec((P2, H, W), lambda k: (k, 0, 0)),
        out_shape=jax.ShapeDtypeStruct((PLANES, H, W), jnp.float32),
    )(cnt, y)
    return out.reshape(b, ch, h, w)


# speedup vs baseline: 2.1054x; 1.0308x over previous
import jax
import jax.numpy as jnp
from jax import lax
from jax.experimental import pallas as pl

BS = 7
H = 224
W = 224
HM = H - (BS - 1)
WM = W - (BS - 1)
PLANES = 384
COUNT_M = float(PLANES * H * W)
P1 = 16
P2 = 32


def _threefry_bits(x1):
    ks0 = jnp.int32(0)
    ks1 = jnp.int32(42)
    ks2 = jnp.int32(42 ^ 0x1BD11BDA)
    x0 = jnp.zeros_like(x1)

    def rotl(v, d):
        return lax.shift_left(v, jnp.int32(d)) | lax.shift_right_logical(
            v, jnp.int32(32 - d)
        )

    rot_a = (13, 15, 26, 6)
    rot_b = (17, 29, 16, 24)
    inject = ((ks1, ks2, 1), (ks2, ks0, 2), (ks0, ks1, 3), (ks1, ks2, 4),
              (ks2, ks0, 5))
    for g in range(5):
        for d in rot_a if g % 2 == 0 else rot_b:
            x0 = x0 + x1
            x1 = rotl(x1, d)
            x1 = x1 ^ x0
        a, b, inc = inject[g]
        x0 = x0 + a
        x1 = x1 + b + jnp.int32(inc)
    return x0 ^ x1


def _phase1_kernel(gamma_ref, idx_ref, vmask_ref, lb_ref, rb_ref, x_ref,
                   y_ref, cnt_ref):
    k = pl.program_id(0)
    gamma = gamma_ref[...]
    base = k * jnp.int32(P1 * HM * WM)
    bits = _threefry_bits(idx_ref[...] + base)
    u = lax.bitcast_convert_type(
        lax.shift_right_logical(bits, jnp.int32(9)) | jnp.int32(0x3F800000),
        jnp.float32,
    ) - jnp.float32(1.0)
    seed = jnp.where(u < gamma, vmask_ref[...], jnp.float32(0.0))
    sb = seed.astype(jnp.bfloat16)
    lb = lb_ref[...]
    rb = rb_ref[...]
    ones = None
    for p in range(P1):
        t = jnp.dot(lb, sb[p], preferred_element_type=jnp.float32)
        d = jnp.dot(t.astype(jnp.bfloat16), rb,
                    preferred_element_type=jnp.float32)
        bm = jnp.where(d < jnp.float32(0.5), jnp.float32(1.0), jnp.float32(0.0))
        y_ref[p, :, :] = (bm * x_ref[p, :, :]).astype(jnp.bfloat16)
        s = jnp.sum(bm, keepdims=True).reshape(1, 1)
        ones = s if ones is None else ones + s

    @pl.when(k == 0)
    def _():
        cnt_ref[...] = jnp.zeros((1, 1), jnp.float32)

    cnt_ref[...] += ones


def _phase2_kernel(cnt_ref, y_ref, out_ref):
    scale = jnp.float32(COUNT_M) / cnt_ref[...]
    out_ref[...] = y_ref[...].astype(jnp.float32) * scale


def kernel(x, gamma):
    b, ch, h, w = x.shape
    xf = x.reshape(b * ch, h, w)
    gamma_arr = jnp.asarray(gamma, jnp.float32).reshape(1, 1)

    pp = jnp.arange(P1, dtype=jnp.int32)[:, None, None]
    rr = jnp.arange(H, dtype=jnp.int32)[None, :, None]
    cc = jnp.arange(W, dtype=jnp.int32)[None, None, :]
    idx0 = pp * (HM * WM) + rr * WM + cc + 42
    vmask = jnp.where((rr < HM) & (cc < WM), jnp.float32(1.0),
                      jnp.float32(0.0)) + jnp.zeros((P1, 1, 1), jnp.float32)
    ii = jnp.arange(H, dtype=jnp.int32)
    band = ((ii[None, :] <= ii[:, None]) &
            (ii[:, None] - ii[None, :] <= BS - 1))
    lband = band.astype(jnp.bfloat16)
    rband = lband.T

    y, cnt = pl.pallas_call(
        _phase1_kernel,
        grid=(PLANES // P1,),
        in_specs=[
            pl.BlockSpec((1, 1), lambda k: (0, 0)),
            pl.BlockSpec((P1, H, W), lambda k: (0, 0, 0)),
            pl.BlockSpec((P1, H, W), lambda k: (0, 0, 0)),
            pl.BlockSpec((H, W), lambda k: (0, 0)),
            pl.BlockSpec((H, W), lambda k: (0, 0)),
            pl.BlockSpec((P1, H, W), lambda k: (k, 0, 0)),
        ],
        out_specs=[
            pl.BlockSpec((P1, H, W), lambda k: (k, 0, 0)),
            pl.BlockSpec((1, 1), lambda k: (0, 0)),
        ],
        out_shape=[
            jax.ShapeDtypeStruct((PLANES, H, W), jnp.bfloat16),
            jax.ShapeDtypeStruct((1, 1), jnp.float32),
        ],
    )(gamma_arr, idx0, vmask, lband, rband, xf)
    out = pl.pallas_call(
        _phase2_kernel,
        grid=(PLANES // P2,),
        in_specs=[
            pl.BlockSpec((1, 1), lambda k: (0, 0)),
            pl.BlockSpec((P2, H, W), lambda k: (k, 0, 0)),
        ],
        out_specs=pl.BlockSpec((P2, H, W), lambda k: (k, 0, 0)),
        out_shape=jax.ShapeDtypeStruct((PLANES, H, W), jnp.float32),
    )(cnt, y)
    return out.reshape(b, ch, h, w)
